# Initial kernel scaffold; baseline (speedup 1.0000x reference)
import functools

import jax
import jax.numpy as jnp
from jax import lax
from jax.experimental import pallas as pl
from jax.experimental.pallas import tpu as pltpu
from jax.experimental.pallas import tpu_sc as plsc

NC = 2
NS = 16
NW = NC * NS


def _sc_mesh():
    return plsc.VectorSubcoreMesh(
        core_axis_name="c", subcore_axis_name="s", num_cores=NC, num_subcores=NS
    )


def _sc_gather(table, idx, chunk):
    E = idx.shape[0]
    D = table.shape[1]
    per_w = E // NW
    n_chunks = per_w // chunk
    assert per_w % chunk == 0 and per_w % 8 == 0 and chunk % 8 == 0

    @functools.partial(
        pl.kernel,
        out_type=jax.ShapeDtypeStruct((E, D), jnp.float32),
        mesh=_sc_mesh(),
        scratch_types=[
            pltpu.VMEM((chunk,), jnp.int32),
            pltpu.VMEM((chunk, D), jnp.float32),
            pltpu.SemaphoreType.DMA,
        ],
    )
    def k(table_hbm, idx_hbm, out_hbm, idx_v, rows_v, sem):
        wid = lax.axis_index("s") * NC + lax.axis_index("c")
        base = wid * per_w
        for j in range(n_chunks):
            off = base + j * chunk
            pltpu.sync_copy(idx_hbm.at[pl.ds(off, chunk)], idx_v)
            pltpu.async_copy(table_hbm.at[idx_v], rows_v, sem).wait()
            pltpu.sync_copy(rows_v, out_hbm.at[pl.ds(off, chunk)])

    return k(table, idx)


def _sc_scatter_add(msgs, idx, zeros, chunk):
    E, D = msgs.shape
    N = zeros.shape[0]
    per_w = E // NW
    n_chunks = per_w // chunk
    rows_per_tile = N // NS
    assert per_w % chunk == 0 and chunk % 8 == 0 and N % NS == 0

    @functools.partial(
        pl.kernel,
        out_type=jax.ShapeDtypeStruct((NC, N, D), jnp.float32),
        mesh=_sc_mesh(),
        scratch_types=[
            pltpu.VMEM((chunk,), jnp.int32),
            pltpu.VMEM((chunk, D), jnp.float32),
            pltpu.VMEM_SHARED((N, D), jnp.float32),
            pltpu.SemaphoreType.DMA,
        ],
    )
    def k(m_hbm, idx_hbm, zeros_hbm, out_hbm, idx_v, m_v, acc_sh, sem):
        cid = lax.axis_index("c")
        sid = lax.axis_index("s")
        wid = sid * NC + cid
        r0 = sid * rows_per_tile
        pltpu.sync_copy(
            zeros_hbm.at[pl.ds(r0, rows_per_tile)],
            acc_sh.at[pl.ds(r0, rows_per_tile)],
        )
        plsc.subcore_barrier()
        base = wid * per_w
        for j in range(n_chunks):
            off = base + j * chunk
            pltpu.sync_copy(idx_hbm.at[pl.ds(off, chunk)], idx_v)
            pltpu.sync_copy(m_hbm.at[pl.ds(off, chunk)], m_v)
            pltpu.sync_copy(m_v, acc_sh.at[idx_v], add=True)
        plsc.subcore_barrier()
        pltpu.sync_copy(
            acc_sh.at[pl.ds(r0, rows_per_tile)],
            out_hbm.at[cid, pl.ds(r0, rows_per_tile)],
        )

    return k(msgs, idx, zeros)


def _edge_mlp(Xg, ea, Wcat, wa, ba, with_count):
    E, F = Xg.shape
    H = Wcat.shape[1] // 11
    BE = 4000
    assert E % BE == 0

    def body(xg_ref, ea_ref, wcat_ref, wa_ref, ba_ref, out_ref):
        a = ea_ref[...]
        g = jnp.maximum(a * wa_ref[...] + ba_ref[...], 0.0)
        P = jnp.dot(xg_ref[...], wcat_ref[...], preferred_element_type=jnp.float32)
        m = P[:, 10 * H:]
        for k in range(10):
            m = m + g[:, k:k + 1] * P[:, k * H:(k + 1) * H]
        if with_count:
            out_ref[:, 0:H] = m
            out_ref[:, H:H + 1] = jnp.ones((BE, 1), jnp.float32)
            out_ref[:, H + 1:32] = jnp.zeros((BE, 32 - H - 1), jnp.float32)
        else:
            out_ref[...] = m

    return pl.pallas_call(
        body,
        grid=(E // BE,),
        in_specs=[
            pl.BlockSpec((BE, F), lambda i: (i, 0)),
            pl.BlockSpec((BE, 1), lambda i: (i, 0)),
            pl.BlockSpec(Wcat.shape, lambda i: (0, 0)),
            pl.BlockSpec(wa.shape, lambda i: (0, 0)),
            pl.BlockSpec(ba.shape, lambda i: (0, 0)),
        ],
        out_specs=pl.BlockSpec((BE, 32), lambda i: (i, 0)),
        out_shape=jax.ShapeDtypeStruct((E, 32), jnp.float32),
    )(Xg, ea, Wcat, wa, ba)


def _elu(v):
    return jnp.where(v > 0, v, jnp.exp(jnp.minimum(v, 0.0)) - 1.0)


def _node1(x, aggA, aggB, root1, bias1, H):
    N = x.shape[0]

    def body(x_ref, a_ref, b_ref, r_ref, bias_ref, h1_ref, cnt_ref):
        s = a_ref[:, 0:H] + b_ref[:, 0:H]
        cnt = a_ref[:, H:H + 1] + b_ref[:, H:H + 1]
        agg = s / jnp.maximum(cnt, 1.0)
        xr = jnp.dot(x_ref[...], r_ref[...], preferred_element_type=jnp.float32)
        h1_ref[...] = _elu(xr + agg + bias_ref[...])
        cnt_ref[...] = cnt

    return pl.pallas_call(
        body,
        out_shape=(
            jax.ShapeDtypeStruct((N, H), jnp.float32),
            jax.ShapeDtypeStruct((N, 1), jnp.float32),
        ),
    )(x, aggA, aggB, root1, bias1)


def _node2_head(h1, cnt, aggA, aggB, root2, bias2, batch2d, Wfc1, bfc1, Wfc2,
                bfc2, n_graphs):
    N, _ = h1.shape
    n_cls = Wfc2.shape[1]

    def body(h1_ref, cnt_ref, a_ref, b_ref, r2_ref, bias2_ref, batch_ref,
             w1_ref, c1_ref, w2_ref, c2_ref, out_ref):
        s = a_ref[...] + b_ref[...]
        agg = s / jnp.maximum(cnt_ref[...], 1.0)
        h2 = _elu(
            jnp.dot(h1_ref[...], r2_ref[...], preferred_element_type=jnp.float32)
            + agg + bias2_ref[...])
        gids = lax.broadcasted_iota(jnp.int32, (n_graphs, N), 0)
        mask = (batch_ref[...] == gids).astype(jnp.float32)
        pooled = jnp.dot(mask, h2, preferred_element_type=jnp.float32)
        cg = jnp.maximum(jnp.sum(mask, axis=1, keepdims=True), 1.0)
        pooled = pooled / cg
        z = _elu(jnp.dot(pooled, w1_ref[...], preferred_element_type=jnp.float32)
                 + c1_ref[...])
        logits = (jnp.dot(z, w2_ref[...], preferred_element_type=jnp.float32)
                  + c2_ref[...])
        mx = jnp.max(logits, axis=1, keepdims=True)
        lse = jnp.log(jnp.sum(jnp.exp(logits - mx), axis=1, keepdims=True)) + mx
        out_ref[...] = logits - lse

    return pl.pallas_call(
        body,
        out_shape=jax.ShapeDtypeStruct((n_graphs, n_cls), jnp.float32),
    )(h1, cnt, aggA, aggB, root2, bias2, batch2d, Wfc1, bfc1, Wfc2, bfc2)


def kernel(x, edge_index, edge_attr, batch, W1a, b1a, W1b, b1b, root1, bias1,
           W2a, b2a, W2b, b2b, root2, bias2, Wfc1, bfc1, Wfc2, bfc2):
    N, F = x.shape
    E = edge_attr.shape[0]
    H1 = root1.shape[1]
    H2 = root2.shape[1]
    n_graphs = 64

    src = edge_index[0]
    dst = edge_index[1]
    Wcat1 = jnp.concatenate(
        [W1b.reshape(10, F, H1).transpose(1, 0, 2).reshape(F, 10 * H1),
         b1b.reshape(F, H1)], axis=1)
    Wcat2 = jnp.concatenate(
        [W2b.reshape(10, H1, H2).transpose(1, 0, 2).reshape(H1, 10 * H2),
         b2b.reshape(H1, H2)], axis=1)
    zeros = jnp.zeros((N, 32), jnp.float32)

    Xg = _sc_gather(x, src, chunk=200)
    m1 = _edge_mlp(Xg, edge_attr, Wcat1, W1a, b1a.reshape(1, 10), with_count=True)
    agg1 = _sc_scatter_add(m1, dst, zeros, chunk=1000)
    h1, cnt = _node1(x, agg1[0], agg1[1], root1, bias1.reshape(1, H1), H1)

    Hg = _sc_gather(h1, src, chunk=1000)
    m2 = _edge_mlp(Hg, edge_attr, Wcat2, W2a, b2a.reshape(1, 10), with_count=False)
    agg2 = _sc_scatter_add(m2, dst, zeros, chunk=1000)

    return _node2_head(h1, cnt, agg2[0], agg2[1], root2, bias2.reshape(1, H2),
                       batch.reshape(1, N).astype(jnp.int32),
                       Wfc1, bfc1.reshape(1, -1), Wfc2, bfc2.reshape(1, -1),
                       n_graphs)


# speedup vs baseline: 2.1016x; 2.1016x over previous
import functools

import jax
import jax.numpy as jnp
from jax import lax
from jax.experimental import pallas as pl
from jax.experimental.pallas import tpu as pltpu
from jax.experimental.pallas import tpu_sc as plsc

NC = 2
NS = 16
NW = NC * NS


def _sc_mesh():
    return plsc.VectorSubcoreMesh(
        core_axis_name="c", subcore_axis_name="s", num_cores=NC, num_subcores=NS
    )


def _sc_gather(table, idx, chunk):
    E = idx.shape[0]
    D = table.shape[1]
    per_w = E // NW
    n_chunks = per_w // chunk
    assert per_w % chunk == 0 and per_w % 8 == 0 and chunk % 8 == 0

    @functools.partial(
        pl.kernel,
        out_type=jax.ShapeDtypeStruct((E, D), jnp.float32),
        mesh=_sc_mesh(),
        scratch_types=[
            pltpu.VMEM((chunk,), jnp.int32),
            pltpu.VMEM((chunk, D), jnp.float32),
            pltpu.SemaphoreType.DMA,
        ],
        compiler_params=pltpu.CompilerParams(use_tc_tiling_on_sc=False),
    )
    def k(table_hbm, idx_hbm, out_hbm, idx_v, rows_v, sem):
        wid = lax.axis_index("s") * NC + lax.axis_index("c")
        base = wid * per_w
        for j in range(n_chunks):
            off = base + j * chunk
            pltpu.sync_copy(idx_hbm.at[pl.ds(off, chunk)], idx_v)
            pltpu.async_copy(table_hbm.at[idx_v], rows_v, sem).wait()
            pltpu.sync_copy(rows_v, out_hbm.at[pl.ds(off, chunk)])

    return k(table, idx)


def _sc_scatter_add(msgs, idx, zeros, chunk):
    E, D = msgs.shape
    N = zeros.shape[0]
    per_w = E // NW
    n_chunks = per_w // chunk
    rows_per_tile = N // NS
    assert per_w % chunk == 0 and chunk % 8 == 0
    assert N % NS == 0 and rows_per_tile % 8 == 0

    @functools.partial(
        pl.kernel,
        out_type=jax.ShapeDtypeStruct((NC, N, D), jnp.float32),
        mesh=_sc_mesh(),
        scratch_types=[
            pltpu.VMEM((chunk,), jnp.int32),
            pltpu.VMEM((chunk, D), jnp.float32),
            pltpu.VMEM_SHARED((N, D), jnp.float32),
            pltpu.SemaphoreType.DMA,
        ],
        compiler_params=pltpu.CompilerParams(use_tc_tiling_on_sc=False),
    )
    def k(m_hbm, idx_hbm, zeros_hbm, out_hbm, idx_v, m_v, acc_sh, sem):
        cid = lax.axis_index("c")
        sid = lax.axis_index("s")
        wid = sid * NC + cid
        r0 = sid * rows_per_tile
        pltpu.sync_copy(
            zeros_hbm.at[pl.ds(r0, rows_per_tile)],
            acc_sh.at[pl.ds(r0, rows_per_tile)],
        )
        plsc.subcore_barrier()
        base = wid * per_w
        for j in range(n_chunks):
            off = base + j * chunk
            pltpu.sync_copy(idx_hbm.at[pl.ds(off, chunk)], idx_v)
            pltpu.sync_copy(m_hbm.at[pl.ds(off, chunk)], m_v)
            pltpu.sync_copy(m_v, acc_sh.at[idx_v], add=True)
        plsc.subcore_barrier()
        pltpu.sync_copy(
            acc_sh.at[pl.ds(r0, rows_per_tile)],
            out_hbm.at[cid, pl.ds(r0, rows_per_tile)],
        )

    return k(msgs, idx, zeros)


def _edge_mlp(Xg, ea, Wcat, wa, ba, with_count):
    E, F = Xg.shape
    H = Wcat.shape[1] // 11
    BE = 4000
    assert E % BE == 0

    def body(xg_ref, ea_ref, wcat_ref, wa_ref, ba_ref, out_ref):
        a = ea_ref[...]
        g = jnp.maximum(a * wa_ref[...] + ba_ref[...], 0.0)
        P = jnp.dot(xg_ref[...], wcat_ref[...], preferred_element_type=jnp.float32)
        m = P[:, 10 * H:]
        for k in range(10):
            m = m + g[:, k:k + 1] * P[:, k * H:(k + 1) * H]
        if with_count:
            out_ref[:, 0:H] = m
            out_ref[:, H:H + 1] = jnp.ones((BE, 1), jnp.float32)
            out_ref[:, H + 1:32] = jnp.zeros((BE, 32 - H - 1), jnp.float32)
        else:
            out_ref[...] = m

    return pl.pallas_call(
        body,
        grid=(E // BE,),
        in_specs=[
            pl.BlockSpec((BE, F), lambda i: (i, 0)),
            pl.BlockSpec((BE, 1), lambda i: (i, 0)),
            pl.BlockSpec(Wcat.shape, lambda i: (0, 0)),
            pl.BlockSpec(wa.shape, lambda i: (0, 0)),
            pl.BlockSpec(ba.shape, lambda i: (0, 0)),
        ],
        out_specs=pl.BlockSpec((BE, 32), lambda i: (i, 0)),
        out_shape=jax.ShapeDtypeStruct((E, 32), jnp.float32),
    )(Xg, ea, Wcat, wa, ba)


def _elu(v):
    return jnp.where(v > 0, v, jnp.exp(jnp.minimum(v, 0.0)) - 1.0)


def _node1(x, aggA, aggB, root1, bias1, H):
    N = x.shape[0]

    def body(x_ref, a_ref, b_ref, r_ref, bias_ref, h1_ref, cnt_ref):
        s = a_ref[:, 0:H] + b_ref[:, 0:H]
        cnt = a_ref[:, H:H + 1] + b_ref[:, H:H + 1]
        agg = s / jnp.maximum(cnt, 1.0)
        xr = jnp.dot(x_ref[...], r_ref[...], preferred_element_type=jnp.float32)
        h1_ref[...] = _elu(xr + agg + bias_ref[...])
        cnt_ref[...] = cnt

    return pl.pallas_call(
        body,
        out_shape=(
            jax.ShapeDtypeStruct((N, H), jnp.float32),
            jax.ShapeDtypeStruct((N, 1), jnp.float32),
        ),
    )(x, aggA, aggB, root1, bias1)


def _node2_head(h1, cnt, aggA, aggB, root2, bias2, batch2d, Wfc1, bfc1, Wfc2,
                bfc2, n_graphs):
    N, _ = h1.shape
    n_cls = Wfc2.shape[1]

    def body(h1_ref, cnt_ref, a_ref, b_ref, r2_ref, bias2_ref, batch_ref,
             w1_ref, c1_ref, w2_ref, c2_ref, out_ref):
        s = a_ref[...] + b_ref[...]
        agg = s / jnp.maximum(cnt_ref[...], 1.0)
        h2 = _elu(
            jnp.dot(h1_ref[...], r2_ref[...], preferred_element_type=jnp.float32)
            + agg + bias2_ref[...])
        gids = lax.broadcasted_iota(jnp.int32, (n_graphs, N), 0)
        mask = (batch_ref[...] == gids).astype(jnp.float32)
        pooled = jnp.dot(mask, h2, preferred_element_type=jnp.float32)
        cg = jnp.maximum(jnp.sum(mask, axis=1, keepdims=True), 1.0)
        pooled = pooled / cg
        z = _elu(jnp.dot(pooled, w1_ref[...], preferred_element_type=jnp.float32)
                 + c1_ref[...])
        logits = (jnp.dot(z, w2_ref[...], preferred_element_type=jnp.float32)
                  + c2_ref[...])
        mx = jnp.max(logits, axis=1, keepdims=True)
        lse = jnp.log(jnp.sum(jnp.exp(logits - mx), axis=1, keepdims=True)) + mx
        out_ref[...] = logits - lse

    return pl.pallas_call(
        body,
        out_shape=jax.ShapeDtypeStruct((n_graphs, n_cls), jnp.float32),
    )(h1, cnt, aggA, aggB, root2, bias2, batch2d, Wfc1, bfc1, Wfc2, bfc2)


def kernel(x, edge_index, edge_attr, batch, W1a, b1a, W1b, b1b, root1, bias1,
           W2a, b2a, W2b, b2b, root2, bias2, Wfc1, bfc1, Wfc2, bfc2):
    N, F = x.shape
    E = edge_attr.shape[0]
    H1 = root1.shape[1]
    H2 = root2.shape[1]
    n_graphs = 64

    src = edge_index[0]
    dst = edge_index[1]
    Wcat1 = jnp.concatenate(
        [W1b.reshape(10, F, H1).transpose(1, 0, 2).reshape(F, 10 * H1),
         b1b.reshape(F, H1)], axis=1)
    Wcat2 = jnp.concatenate(
        [W2b.reshape(10, H1, H2).transpose(1, 0, 2).reshape(H1, 10 * H2),
         b2b.reshape(H1, H2)], axis=1)
    n_pad = ((N + NS * 8 - 1) // (NS * 8)) * (NS * 8)
    zeros = jnp.zeros((n_pad, 32), jnp.float32)

    Xg = _sc_gather(x, src, chunk=200)
    m1 = _edge_mlp(Xg, edge_attr, Wcat1, W1a, b1a.reshape(1, 10), with_count=True)
    agg1 = _sc_scatter_add(m1, dst, zeros, chunk=200)
    h1, cnt = _node1(x, agg1[0, :N], agg1[1, :N], root1, bias1.reshape(1, H1), H1)

    Hg = _sc_gather(h1, src, chunk=200)
    m2 = _edge_mlp(Hg, edge_attr, Wcat2, W2a, b2a.reshape(1, 10), with_count=False)
    agg2 = _sc_scatter_add(m2, dst, zeros, chunk=200)

    return _node2_head(h1, cnt, agg2[0, :N], agg2[1, :N], root2,
                       bias2.reshape(1, H2),
                       batch.reshape(1, N).astype(jnp.int32),
                       Wfc1, bfc1.reshape(1, -1), Wfc2, bfc2.reshape(1, -1),
                       n_graphs)


# speedup vs baseline: 4.2778x; 2.0355x over previous
import functools

import jax
import jax.numpy as jnp
from jax import lax
from jax.experimental import pallas as pl
from jax.experimental.pallas import tpu as pltpu
from jax.experimental.pallas import tpu_sc as plsc

NC = 2
NS = 16
NW = NC * NS


def _sc_mesh():
    return plsc.VectorSubcoreMesh(
        core_axis_name="c", subcore_axis_name="s", num_cores=NC, num_subcores=NS
    )


def _sc_gather(table, idx, chunk):
    E = idx.shape[0]
    D = table.shape[1]
    per_w = E // NW
    n_chunks = per_w // chunk
    assert per_w % chunk == 0 and per_w % 8 == 0 and chunk % 8 == 0

    @functools.partial(
        pl.kernel,
        out_type=jax.ShapeDtypeStruct((E, D), jnp.float32),
        mesh=_sc_mesh(),
        scratch_types=[
            pltpu.VMEM((chunk,), jnp.int32),
            pltpu.VMEM((chunk, D), jnp.float32),
            pltpu.SemaphoreType.DMA,
        ],
        compiler_params=pltpu.CompilerParams(use_tc_tiling_on_sc=(D % 128 == 0)),
    )
    def k(table_hbm, idx_hbm, out_hbm, idx_v, rows_v, sem):
        wid = lax.axis_index("s") * NC + lax.axis_index("c")
        base = wid * per_w
        for j in range(n_chunks):
            off = base + j * chunk
            pltpu.sync_copy(idx_hbm.at[pl.ds(off, chunk)], idx_v)
            pltpu.async_copy(table_hbm.at[idx_v], rows_v, sem).wait()
            pltpu.sync_copy(rows_v, out_hbm.at[pl.ds(off, chunk)])

    return k(table, idx)


def _sc_scatter_add(msgs, idx, zeros, chunk):
    E, D = msgs.shape
    N = zeros.shape[0]
    per_w = E // NW
    n_chunks = per_w // chunk
    rows_per_tile = N // NS
    assert per_w % chunk == 0 and chunk % 8 == 0
    assert N % NS == 0 and rows_per_tile % 8 == 0

    @functools.partial(
        pl.kernel,
        out_type=jax.ShapeDtypeStruct((NC, N, D), jnp.float32),
        mesh=_sc_mesh(),
        scratch_types=[
            pltpu.VMEM((chunk,), jnp.int32),
            pltpu.VMEM((chunk, D), jnp.float32),
            pltpu.VMEM_SHARED((N, D), jnp.float32),
            pltpu.SemaphoreType.DMA,
        ],
        compiler_params=pltpu.CompilerParams(use_tc_tiling_on_sc=False),
    )
    def k(m_hbm, idx_hbm, zeros_hbm, out_hbm, idx_v, m_v, acc_sh, sem):
        cid = lax.axis_index("c")
        sid = lax.axis_index("s")
        wid = sid * NC + cid
        r0 = sid * rows_per_tile
        pltpu.sync_copy(
            zeros_hbm.at[pl.ds(r0, rows_per_tile)],
            acc_sh.at[pl.ds(r0, rows_per_tile)],
        )
        plsc.subcore_barrier()
        base = wid * per_w
        for j in range(n_chunks):
            off = base + j * chunk
            pltpu.sync_copy(idx_hbm.at[pl.ds(off, chunk)], idx_v)
            pltpu.sync_copy(m_hbm.at[pl.ds(off, chunk)], m_v)
            pltpu.sync_copy(m_v, acc_sh.at[idx_v], add=True)
        plsc.subcore_barrier()
        pltpu.sync_copy(
            acc_sh.at[pl.ds(r0, rows_per_tile)],
            out_hbm.at[cid, pl.ds(r0, rows_per_tile)],
        )

    return k(msgs, idx, zeros)


def _edge_mlp(Xg, ea, Wcatp, wat, bat, S, crow):
    E, F = Xg.shape
    BE = 4000
    assert E % BE == 0

    def body(xg_ref, ea_ref, wcat_ref, wat_ref, bat_ref, s_ref, crow_ref,
             out_ref):
        a = ea_ref[...]
        ct = jnp.maximum(a * wat_ref[...] + bat_ref[...], 0.0)
        P = jnp.dot(xg_ref[...], wcat_ref[...], preferred_element_type=jnp.float32)
        out = jnp.dot(ct * P, s_ref[...], preferred_element_type=jnp.float32)
        out_ref[...] = out + crow_ref[...]

    return pl.pallas_call(
        body,
        grid=(E // BE,),
        in_specs=[
            pl.BlockSpec((BE, F), lambda i: (i, 0)),
            pl.BlockSpec((BE, 1), lambda i: (i, 0)),
            pl.BlockSpec(Wcatp.shape, lambda i: (0, 0)),
            pl.BlockSpec(wat.shape, lambda i: (0, 0)),
            pl.BlockSpec(bat.shape, lambda i: (0, 0)),
            pl.BlockSpec(S.shape, lambda i: (0, 0)),
            pl.BlockSpec(crow.shape, lambda i: (0, 0)),
        ],
        out_specs=pl.BlockSpec((BE, 32), lambda i: (i, 0)),
        out_shape=jax.ShapeDtypeStruct((E, 32), jnp.float32),
    )(Xg, ea, Wcatp, wat, bat, S, crow)


def _elu(v):
    return jnp.where(v > 0, v, jnp.exp(jnp.minimum(v, 0.0)) - 1.0)


def _node1(x, aggA, aggB, root1, bias1, H):
    N = x.shape[0]

    def body(x_ref, a_ref, b_ref, r_ref, bias_ref, h1_ref, cnt_ref):
        s = a_ref[:, 0:H] + b_ref[:, 0:H]
        cnt = a_ref[:, H:H + 1] + b_ref[:, H:H + 1]
        agg = s / jnp.maximum(cnt, 1.0)
        xr = jnp.dot(x_ref[...], r_ref[...], preferred_element_type=jnp.float32)
        h1_ref[...] = _elu(xr + agg + bias_ref[...])
        cnt_ref[...] = cnt

    return pl.pallas_call(
        body,
        out_shape=(
            jax.ShapeDtypeStruct((N, H), jnp.float32),
            jax.ShapeDtypeStruct((N, 1), jnp.float32),
        ),
    )(x, aggA, aggB, root1, bias1)


def _node2_head(h1, cnt, aggA, aggB, root2, bias2, batch2d, Wfc1, bfc1, Wfc2,
                bfc2, n_graphs):
    N, _ = h1.shape
    n_cls = Wfc2.shape[1]

    def body(h1_ref, cnt_ref, a_ref, b_ref, r2_ref, bias2_ref, batch_ref,
             w1_ref, c1_ref, w2_ref, c2_ref, out_ref):
        s = a_ref[...] + b_ref[...]
        agg = s / jnp.maximum(cnt_ref[...], 1.0)
        h2 = _elu(
            jnp.dot(h1_ref[...], r2_ref[...], preferred_element_type=jnp.float32)
            + agg + bias2_ref[...])
        gids = lax.broadcasted_iota(jnp.int32, (n_graphs, N), 0)
        mask = (batch_ref[...] == gids).astype(jnp.float32)
        pooled = jnp.dot(mask, h2, preferred_element_type=jnp.float32)
        cg = jnp.maximum(jnp.sum(mask, axis=1, keepdims=True), 1.0)
        pooled = pooled / cg
        z = _elu(jnp.dot(pooled, w1_ref[...], preferred_element_type=jnp.float32)
                 + c1_ref[...])
        logits = (jnp.dot(z, w2_ref[...], preferred_element_type=jnp.float32)
                  + c2_ref[...])
        mx = jnp.max(logits, axis=1, keepdims=True)
        lse = jnp.log(jnp.sum(jnp.exp(logits - mx), axis=1, keepdims=True)) + mx
        out_ref[...] = logits - lse

    return pl.pallas_call(
        body,
        out_shape=jax.ShapeDtypeStruct((n_graphs, n_cls), jnp.float32),
    )(h1, cnt, aggA, aggB, root2, bias2, batch2d, Wfc1, bfc1, Wfc2, bfc2)


def kernel(x, edge_index, edge_attr, batch, W1a, b1a, W1b, b1b, root1, bias1,
           W2a, b2a, W2b, b2b, root2, bias2, Wfc1, bfc1, Wfc2, bfc2):
    N, F = x.shape
    E = edge_attr.shape[0]
    H1 = root1.shape[1]
    H2 = root2.shape[1]
    n_graphs = 64

    src = edge_index[0]
    dst = edge_index[1]

    def edge_setup(Wb, bb, Wa, ba, fin, h, with_count):
        B = jnp.concatenate([Wb.reshape(10, fin, h),
                             bb.reshape(1, fin, h)], axis=0)
        Wcatp = B.transpose(1, 2, 0).reshape(fin, h * 11)
        wat = jnp.tile(jnp.concatenate([Wa[0], jnp.zeros((1,))]), h)
        bat = jnp.tile(jnp.concatenate([ba, jnp.ones((1,))]), h)
        S = jnp.concatenate(
            [jnp.kron(jnp.eye(h, dtype=jnp.float32), jnp.ones((11, 1), jnp.float32)),
             jnp.zeros((h * 11, 32 - h), jnp.float32)], axis=1)
        crow = (jax.nn.one_hot(h, 32, dtype=jnp.float32)[None, :]
                if with_count else jnp.zeros((1, 32), jnp.float32))
        return (Wcatp, wat.reshape(1, -1).astype(jnp.float32),
                bat.reshape(1, -1).astype(jnp.float32), S, crow)

    Wcat1p, wat1, bat1, S1, crow1 = edge_setup(W1b, b1b, W1a, b1a, F, H1, True)
    Wcat2p, wat2, bat2, S2, crow2 = edge_setup(W2b, b2b, W2a, b2a, H1, H2, False)
    n_pad = ((N + NS * 8 - 1) // (NS * 8)) * (NS * 8)
    zeros = jnp.zeros((n_pad, 32), jnp.float32)

    Xg = _sc_gather(x, src, chunk=200)
    m1 = _edge_mlp(Xg, edge_attr, Wcat1p, wat1, bat1, S1, crow1)
    agg1 = _sc_scatter_add(m1, dst, zeros, chunk=200)
    h1, cnt = _node1(x, agg1[0, :N], agg1[1, :N], root1, bias1.reshape(1, H1), H1)

    Hg = _sc_gather(h1, src, chunk=200)
    m2 = _edge_mlp(Hg, edge_attr, Wcat2p, wat2, bat2, S2, crow2)
    agg2 = _sc_scatter_add(m2, dst, zeros, chunk=200)

    return _node2_head(h1, cnt, agg2[0, :N], agg2[1, :N], root2,
                       bias2.reshape(1, H2),
                       batch.reshape(1, N).astype(jnp.int32),
                       Wfc1, bfc1.reshape(1, -1), Wfc2, bfc2.reshape(1, -1),
                       n_graphs)
